# Initial kernel scaffold; baseline (speedup 1.0000x reference)
import jax
import jax.numpy as jnp
from jax.experimental import pallas as pl


def kernel(x, edge_index, batch, W0, b0, g0, be0, W1, b1, g1, be1, W2, b2, g2, be2, W3, b3, g3, be3, Wjk, bjk, Wlin, blin):
    raise NotImplementedError("write your pallas kernel here")


# speedup vs baseline: 7.2056x; 7.2056x over previous
import functools

import jax
import jax.numpy as jnp
from jax import lax
from jax.experimental import pallas as pl
from jax.experimental.pallas import tpu as pltpu
from jax.experimental.pallas import tpu_sc as plsc

NC = 2
NS = 16
LAN = 16
NW = NC * NS

N = 10000
E = 320000
G = 256
H = 185
HP = 192
NP = 10112
RB = 1264
NRB = NP // RB
NPS = NP // NS
HW = HP // 2
CH = 128
CPW = 79
EPW = CH * CPW
EP = EPW * NW
CPS = 158
EPS = CH * CPS
F32 = jnp.float32
I32 = jnp.int32

_mesh = functools.partial(plsc.VectorSubcoreMesh, core_axis_name="c",
                          subcore_axis_name="s")
_SC_PARAMS = pltpu.CompilerParams(use_tc_tiling_on_sc=False,
                                  needs_layout_passes=False)


def _sc_degree(dstp, zeros_rows, ones_rows):
    @functools.partial(
        pl.kernel,
        out_type=jax.ShapeDtypeStruct((NC, NP, LAN), F32),
        mesh=_mesh(),
        compiler_params=_SC_PARAMS,
        scratch_types=[
            pltpu.VMEM((CH,), I32),
            pltpu.VMEM((CH, LAN), F32),
            pltpu.VMEM_SHARED((NP, LAN), F32),
        ],
    )
    def k(dst_hbm, zero_hbm, ones_hbm, out_hbm, didx, ones_v, acc):
        c = lax.axis_index("c")
        s = lax.axis_index("s")
        w = s * NC + c
        base = s * NPS
        pltpu.sync_copy(ones_hbm, ones_v)
        for j in range(4):
            pltpu.sync_copy(zero_hbm.at[pl.ds(0, CH)],
                            acc.at[pl.ds(base + j * CH, CH)])
        pltpu.sync_copy(zero_hbm.at[pl.ds(0, NPS - 4 * CH)],
                        acc.at[pl.ds(base + 4 * CH, NPS - 4 * CH)])
        plsc.subcore_barrier()
        woff = w * EPW

        @pl.loop(0, CPW)
        def _(i):
            pltpu.sync_copy(dst_hbm.at[pl.ds(woff + i * CH, CH)], didx)
            pltpu.sync_copy(ones_v, acc.at[didx], add=True)

        plsc.subcore_barrier()
        pltpu.sync_copy(acc.at[pl.ds(base, NPS)], out_hbm.at[c, pl.ds(base, NPS)])

    return k(dstp, zeros_rows, ones_rows)


def _sc_aggregate(hs0, hs1, srcp, dstp):
    @functools.partial(
        pl.kernel,
        out_type=jax.ShapeDtypeStruct((NC, NP, HW), F32),
        mesh=_mesh(),
        compiler_params=_SC_PARAMS,
        scratch_types=[
            pltpu.VMEM((CH,), I32),
            pltpu.VMEM((CH,), I32),
            pltpu.VMEM((CH, HW), F32),
            pltpu.VMEM_SHARED((NP, HW), F32),
            pltpu.SemaphoreType.DMA,
        ],
    )
    def k(hs0_hbm, hs1_hbm, src_hbm, dst_hbm, out_hbm, sidx, didx, rows, acc,
          gsem):
        c = lax.axis_index("c")
        s = lax.axis_index("s")
        base = s * NPS
        woff = s * EPS

        def run(h_hbm):
            pltpu.sync_copy(h_hbm.at[pl.ds(base, NPS)], acc.at[pl.ds(base, NPS)])
            plsc.subcore_barrier()

            @pl.loop(0, CPS)
            def _(i):
                eb = woff + i * CH
                pltpu.sync_copy(src_hbm.at[pl.ds(eb, CH)], sidx)
                pltpu.sync_copy(dst_hbm.at[pl.ds(eb, CH)], didx)
                pltpu.async_copy(h_hbm.at[sidx], rows, gsem).wait()
                pltpu.sync_copy(rows, acc.at[didx], add=True)

            plsc.subcore_barrier()
            pltpu.sync_copy(acc.at[pl.ds(base, NPS)],
                            out_hbm.at[c, pl.ds(base, NPS)])

        @pl.when(c == 0)
        def _():
            run(hs0_hbm)

        @pl.when(c == 1)
        def _():
            run(hs1_hbm)

    return k(hs0, hs1, srcp, dstp)


def _sc_segmax(hjk, starts):
    NCB = HP // LAN

    @functools.partial(
        pl.kernel,
        out_type=jax.ShapeDtypeStruct((G, HP), F32),
        mesh=_mesh(),
        compiler_params=_SC_PARAMS,
        scratch_types=[
            pltpu.VMEM((LAN,), I32),
            pltpu.VMEM((LAN, HP), F32),
            pltpu.VMEM((1, HP), F32),
        ],
    )
    def k(hjk_hbm, starts_hbm, out_hbm, stv, buf, orow):
        c = lax.axis_index("c")
        s = lax.axis_index("s")
        w = s * NC + c
        pltpu.sync_copy(starts_hbm.at[w], stv)
        lane = lax.iota(I32, LAN)
        sv = stv[...]

        def getscal(j):
            return jnp.max(jnp.where(lane == j, sv, 0))

        for j in range(8):
            glo = getscal(j)
            ghi = getscal(j + 1)
            cbase = jnp.bitwise_and(glo, -LAN)
            nch = (ghi - cbase + (LAN - 1)) // LAN
            neg = jnp.full((LAN,), -jnp.inf, F32)
            acc0 = tuple(neg for _ in range(NCB))

            def chunk(kk, accs, glo=glo, ghi=ghi, cbase=cbase):
                c0 = cbase + kk * LAN
                pltpu.sync_copy(hjk_hbm.at[pl.ds(c0, LAN)], buf)
                out = list(accs)
                for r in range(LAN):
                    valid = jnp.logical_and(c0 + r >= glo, c0 + r < ghi)
                    for cb in range(NCB):
                        v = buf[r, pl.ds(cb * LAN, LAN)]
                        v = jnp.where(valid, v, neg)
                        out[cb] = jnp.maximum(out[cb], v)
                return tuple(out)

            accs = lax.fori_loop(0, nch, chunk, acc0)
            for cb in range(NCB):
                orow[0, pl.ds(cb * LAN, LAN)] = accs[cb]
            g = w * 8 + j
            pltpu.sync_copy(orow, out_hbm.at[pl.ds(g, 1)])

    return k(hjk, starts)


def _tc_dinv(degp):
    def body(dp_ref, o_ref):
        d = dp_ref[0] + dp_ref[1]
        deg = d[:, 0:1] + 1.0
        o_ref[...] = lax.rsqrt(deg)

    return pl.pallas_call(
        body,
        out_shape=jax.ShapeDtypeStruct((NP, 1), F32),
    )(degp)


def _tc_matmul(h_in, W, dinv, affine=None):
    K = h_in.shape[1]

    def body(h_ref, w_ref, dv_ref, *rest):
        if affine is None:
            o0_ref, o1_ref = rest
            a = h_ref[...]
        else:
            s_ref, t_ref, o0_ref, o1_ref = rest
            a = jnp.maximum(h_ref[...] * s_ref[...] + t_ref[...], 0.0)
        mm = jax.lax.dot_general(a, w_ref[...], (((1,), (0,)), ((), ())),
                                 preferred_element_type=F32,
                                 precision=lax.Precision.HIGHEST)
        hs = mm * dv_ref[...]
        o0_ref[...] = hs[:, :HW]
        o1_ref[...] = hs[:, HW:]

    in_specs = [
        pl.BlockSpec((RB, K), lambda i: (i, 0)),
        pl.BlockSpec((K, HP), lambda i: (0, 0)),
        pl.BlockSpec((RB, 1), lambda i: (i, 0)),
    ]
    args = [h_in, W, dinv]
    if affine is not None:
        in_specs += [pl.BlockSpec((1, HP), lambda i: (0, 0))] * 2
        args += [affine[0], affine[1]]
    return pl.pallas_call(
        body,
        grid=(NRB,),
        in_specs=in_specs,
        out_specs=[pl.BlockSpec((RB, HW), lambda i: (i, 0))] * 2,
        out_shape=[jax.ShapeDtypeStruct((NP, HW), F32)] * 2,
    )(*args)


def _tc_reduce(P, dinv, b, g, be):
    def body(p_ref, dv_ref, b_ref, g_ref, be_ref,
             u_ref, s_ref, t_ref, sum_ref, sq_ref):
        i = pl.program_id(0)
        agg = jnp.concatenate([p_ref[0], p_ref[1]], axis=1)
        u = agg * dv_ref[...] + b_ref[...]
        u_ref[...] = u
        rows = lax.broadcasted_iota(I32, (RB, 1), 0) + i * RB
        um = jnp.where(rows < N, u, 0.0)

        @pl.when(i == 0)
        def _():
            sum_ref[...] = jnp.zeros_like(sum_ref)
            sq_ref[...] = jnp.zeros_like(sq_ref)

        sum_ref[...] += jnp.sum(um, axis=0, keepdims=True)
        sq_ref[...] += jnp.sum(um * um, axis=0, keepdims=True)

        @pl.when(i == NRB - 1)
        def _():
            m = sum_ref[...] / N
            v = sq_ref[...] / N - m * m
            sf = g_ref[...] * lax.rsqrt(v + 1e-5)
            s_ref[...] = sf
            t_ref[...] = be_ref[...] - m * sf

    return pl.pallas_call(
        body,
        grid=(NRB,),
        in_specs=[
            pl.BlockSpec((NC, RB, HW), lambda i: (0, i, 0)),
            pl.BlockSpec((RB, 1), lambda i: (i, 0)),
            pl.BlockSpec((1, HP), lambda i: (0, 0)),
            pl.BlockSpec((1, HP), lambda i: (0, 0)),
            pl.BlockSpec((1, HP), lambda i: (0, 0)),
        ],
        out_specs=[
            pl.BlockSpec((RB, HP), lambda i: (i, 0)),
            pl.BlockSpec((1, HP), lambda i: (0, 0)),
            pl.BlockSpec((1, HP), lambda i: (0, 0)),
        ],
        out_shape=[
            jax.ShapeDtypeStruct((NP, HP), F32),
            jax.ShapeDtypeStruct((1, HP), F32),
            jax.ShapeDtypeStruct((1, HP), F32),
        ],
        scratch_shapes=[pltpu.VMEM((1, HP), F32), pltpu.VMEM((1, HP), F32)],
    )(P, dinv, b, g, be)


def _tc_jk(us, sts, Wjk, bjk):
    def body(u0, u1, u2, u3, s0, t0, s1, t1, s2, t2, s3, t3,
             wjk_ref, bjk_ref, o_ref):
        acc = jnp.broadcast_to(bjk_ref[...], (RB, HP))
        for l, (u_ref, s_ref, t_ref) in enumerate(
                ((u0, s0, t0), (u1, s1, t1), (u2, s2, t2), (u3, s3, t3))):
            a = jnp.maximum(u_ref[...] * s_ref[...] + t_ref[...], 0.0)
            acc = acc + jax.lax.dot_general(
                a, wjk_ref[l], (((1,), (0,)), ((), ())),
                preferred_element_type=F32, precision=lax.Precision.HIGHEST)
        o_ref[...] = acc

    in_specs = [pl.BlockSpec((RB, HP), lambda i: (i, 0))] * 4
    in_specs += [pl.BlockSpec((1, HP), lambda i: (0, 0))] * 8
    in_specs += [pl.BlockSpec((4, HP, HP), lambda i: (0, 0, 0)),
                 pl.BlockSpec((1, HP), lambda i: (0, 0))]
    args = list(us)
    for (s, t) in sts:
        args += [s, t]
    args += [Wjk, bjk]
    return pl.pallas_call(
        body,
        grid=(NRB,),
        in_specs=in_specs,
        out_specs=pl.BlockSpec((RB, HP), lambda i: (i, 0)),
        out_shape=jax.ShapeDtypeStruct((NP, HP), F32),
    )(*args)


def _tc_bounds(batch2):
    NB, BL = batch2.shape

    def body(b_ref, o_ref):
        idx = lax.broadcasted_iota(I32, (NW * LAN, 1), 0)
        th = (idx // LAN) * 8 + idx % LAN
        acc = jnp.zeros((NW * LAN, 1), I32)
        for k in range(NB):
            b = b_ref[pl.ds(k, 1), :]
            acc = acc + jnp.sum(jnp.where(b < th, 1, 0).astype(I32),
                                axis=1, keepdims=True)
        o_ref[...] = acc.reshape(NW, LAN)

    return pl.pallas_call(
        body,
        out_shape=jax.ShapeDtypeStruct((NW, LAN), I32),
    )(batch2)


def _tc_head(pooled, Wlin, blin):
    def body(p_ref, w_ref, b_ref, o_ref):
        o_ref[...] = jax.lax.dot_general(
            p_ref[...], w_ref[...], (((1,), (0,)), ((), ())),
            preferred_element_type=F32,
            precision=lax.Precision.HIGHEST) + b_ref[...]

    return pl.pallas_call(
        body,
        out_shape=jax.ShapeDtypeStruct((G, 1), F32),
    )(pooled, Wlin, blin)


def _pad_rows(a, rows):
    return jnp.pad(a, ((0, rows - a.shape[0]), (0, 0)))


def _pad_feat(v):
    return jnp.pad(v.reshape(1, -1), ((0, 0), (0, HP - v.shape[0])))


def kernel(x, edge_index, batch, W0, b0, g0, be0, W1, b1, g1, be1,
           W2, b2, g2, be2, W3, b3, g3, be3, Wjk, bjk, Wlin, blin):
    src = edge_index[0].astype(I32)
    dst = edge_index[1].astype(I32)
    srcp = jnp.concatenate([src, jnp.zeros((EP - E,), I32)])
    dstp = jnp.concatenate([dst, jnp.full((EP - E,), N, I32)])

    xp = _pad_rows(x, NP)
    W0p = jnp.pad(W0, ((0, 0), (0, HP - H)))
    W1p = jnp.pad(W1, ((0, HP - H), (0, HP - H)))
    W2p = jnp.pad(W2, ((0, HP - H), (0, HP - H)))
    W3p = jnp.pad(W3, ((0, HP - H), (0, HP - H)))
    bs = [_pad_feat(b) for b in (b0, b1, b2, b3)]
    gs = [_pad_feat(g) for g in (g0, g1, g2, g3)]
    bes = [_pad_feat(b) for b in (be0, be1, be2, be3)]
    Wjkp = jnp.pad(Wjk.reshape(4, H, H), ((0, 0), (0, HP - H), (0, HP - H)))
    bjkp = _pad_feat(bjk)
    Wlinp = jnp.pad(Wlin, ((0, HP - H), (0, 0)))
    blinp = blin.reshape(1, 1)

    zeros_rows = jnp.zeros((CH, LAN), F32)
    ones_rows = jnp.ones((CH, LAN), F32)

    degp = _sc_degree(dstp, zeros_rows, ones_rows)
    dinv = _tc_dinv(degp)

    us, sts = [], []
    h_in, W_l, aff = xp, W0p, None
    for l in range(4):
        hs0, hs1 = _tc_matmul(h_in, W_l, dinv, affine=aff)
        P = _sc_aggregate(hs0, hs1, srcp, dstp)
        u, s_l, t_l = _tc_reduce(P, dinv, bs[l], gs[l], bes[l])
        us.append(u)
        sts.append((s_l, t_l))
        if l < 3:
            h_in, W_l, aff = u, (W1p, W2p, W3p)[l], sts[-1]

    hjk = _tc_jk(us, sts, Wjkp, bjkp)
    starts = _tc_bounds(batch.astype(I32).reshape(10, N // 10))
    pooled = _sc_segmax(hjk, starts)
    return _tc_head(pooled, Wlinp, blinp)


# speedup vs baseline: 7.8947x; 1.0956x over previous
import functools

import jax
import jax.numpy as jnp
from jax import lax
from jax.experimental import pallas as pl
from jax.experimental.pallas import tpu as pltpu
from jax.experimental.pallas import tpu_sc as plsc

NC = 2
NS = 16
LAN = 16
NW = NC * NS

N = 10000
E = 320000
G = 256
H = 185
HP = 192
NP = 10112
RB = 1264
NRB = NP // RB
NPS = NP // NS
HW = HP // 2
CH = 128
CPW = 79
EPW = CH * CPW
EP = EPW * NW
CPS = 158
EPS = CH * CPS
F32 = jnp.float32
I32 = jnp.int32

_mesh = functools.partial(plsc.VectorSubcoreMesh, core_axis_name="c",
                          subcore_axis_name="s")
_SC_PARAMS = pltpu.CompilerParams(use_tc_tiling_on_sc=False,
                                  needs_layout_passes=False)


def _sc_degree(dstp, zeros_rows, ones_rows):
    @functools.partial(
        pl.kernel,
        out_type=jax.ShapeDtypeStruct((NC, NP, LAN), F32),
        mesh=_mesh(),
        compiler_params=_SC_PARAMS,
        scratch_types=[
            pltpu.VMEM((CH,), I32),
            pltpu.VMEM((CH, LAN), F32),
            pltpu.VMEM_SHARED((NP, LAN), F32),
        ],
    )
    def k(dst_hbm, zero_hbm, ones_hbm, out_hbm, didx, ones_v, acc):
        c = lax.axis_index("c")
        s = lax.axis_index("s")
        w = s * NC + c
        base = s * NPS
        pltpu.sync_copy(ones_hbm, ones_v)
        for j in range(4):
            pltpu.sync_copy(zero_hbm.at[pl.ds(0, CH)],
                            acc.at[pl.ds(base + j * CH, CH)])
        pltpu.sync_copy(zero_hbm.at[pl.ds(0, NPS - 4 * CH)],
                        acc.at[pl.ds(base + 4 * CH, NPS - 4 * CH)])
        plsc.subcore_barrier()
        woff = w * EPW

        @pl.loop(0, CPW)
        def _(i):
            pltpu.sync_copy(dst_hbm.at[pl.ds(woff + i * CH, CH)], didx)
            pltpu.sync_copy(ones_v, acc.at[didx], add=True)

        plsc.subcore_barrier()
        pltpu.sync_copy(acc.at[pl.ds(base, NPS)], out_hbm.at[c, pl.ds(base, NPS)])

    return k(dstp, zeros_rows, ones_rows)


def _sc_aggregate(hs0, hs1, srcp, dstp):
    @functools.partial(
        pl.kernel,
        out_type=jax.ShapeDtypeStruct((NC, NP, HW), F32),
        mesh=_mesh(),
        compiler_params=_SC_PARAMS,
        scratch_types=[
            pltpu.VMEM((CH,), I32),
            pltpu.VMEM((CH,), I32),
            pltpu.VMEM((CH,), I32),
            pltpu.VMEM((CH,), I32),
            pltpu.VMEM((CH, HW), F32),
            pltpu.VMEM((CH, HW), F32),
            pltpu.VMEM_SHARED((NP, HW), F32),
            pltpu.SemaphoreType.DMA,
            pltpu.SemaphoreType.DMA,
        ],
    )
    def k(hs0_hbm, hs1_hbm, src_hbm, dst_hbm, out_hbm, sidx0, didx0, sidx1,
          didx1, rows0, rows1, acc, gsem0, gsem1):
        c = lax.axis_index("c")
        s = lax.axis_index("s")
        base = s * NPS
        woff = s * EPS

        def run(h_hbm):
            pltpu.sync_copy(h_hbm.at[pl.ds(base, NPS)], acc.at[pl.ds(base, NPS)])
            plsc.subcore_barrier()

            def load_idx(eb, si, di):
                pltpu.sync_copy(src_hbm.at[pl.ds(eb, CH)], si)
                pltpu.sync_copy(dst_hbm.at[pl.ds(eb, CH)], di)

            load_idx(woff, sidx0, didx0)
            pltpu.async_copy(h_hbm.at[sidx0], rows0, gsem0)

            @pl.loop(0, CPS // 2)
            def _(kk):
                i0 = woff + 2 * kk * CH
                load_idx(i0 + CH, sidx1, didx1)
                pltpu.async_copy(h_hbm.at[sidx1], rows1, gsem1)
                pltpu.make_async_copy(h_hbm.at[sidx0], rows0, gsem0).wait()
                pltpu.sync_copy(rows0, acc.at[didx0], add=True)

                @pl.when(kk < CPS // 2 - 1)
                def _():
                    load_idx(i0 + 2 * CH, sidx0, didx0)
                    pltpu.async_copy(h_hbm.at[sidx0], rows0, gsem0)

                pltpu.make_async_copy(h_hbm.at[sidx1], rows1, gsem1).wait()
                pltpu.sync_copy(rows1, acc.at[didx1], add=True)

            plsc.subcore_barrier()
            pltpu.sync_copy(acc.at[pl.ds(base, NPS)],
                            out_hbm.at[c, pl.ds(base, NPS)])

        @pl.when(c == 0)
        def _():
            run(hs0_hbm)

        @pl.when(c == 1)
        def _():
            run(hs1_hbm)

    return k(hs0, hs1, srcp, dstp)


def _sc_segmax(hjk, starts):
    NCB = HP // LAN

    @functools.partial(
        pl.kernel,
        out_type=jax.ShapeDtypeStruct((G, HP), F32),
        mesh=_mesh(),
        compiler_params=_SC_PARAMS,
        scratch_types=[
            pltpu.VMEM((LAN,), I32),
            pltpu.VMEM((LAN, HP), F32),
            pltpu.VMEM((1, HP), F32),
        ],
    )
    def k(hjk_hbm, starts_hbm, out_hbm, stv, buf, orow):
        c = lax.axis_index("c")
        s = lax.axis_index("s")
        w = s * NC + c
        pltpu.sync_copy(starts_hbm.at[w], stv)
        lane = lax.iota(I32, LAN)
        sv = stv[...]

        def getscal(j):
            return jnp.max(jnp.where(lane == j, sv, 0))

        for j in range(8):
            glo = getscal(j)
            ghi = getscal(j + 1)
            cbase = jnp.bitwise_and(glo, -LAN)
            nch = (ghi - cbase + (LAN - 1)) // LAN
            neg = jnp.full((LAN,), -jnp.inf, F32)
            acc0 = tuple(neg for _ in range(NCB))

            def chunk(kk, accs, glo=glo, ghi=ghi, cbase=cbase):
                c0 = cbase + kk * LAN
                pltpu.sync_copy(hjk_hbm.at[pl.ds(c0, LAN)], buf)
                out = list(accs)
                for r in range(LAN):
                    valid = jnp.logical_and(c0 + r >= glo, c0 + r < ghi)
                    for cb in range(NCB):
                        v = buf[r, pl.ds(cb * LAN, LAN)]
                        v = jnp.where(valid, v, neg)
                        out[cb] = jnp.maximum(out[cb], v)
                return tuple(out)

            accs = lax.fori_loop(0, nch, chunk, acc0)
            for cb in range(NCB):
                orow[0, pl.ds(cb * LAN, LAN)] = accs[cb]
            g = w * 8 + j
            pltpu.sync_copy(orow, out_hbm.at[pl.ds(g, 1)])

    return k(hjk, starts)


def _tc_dinv(degp):
    def body(dp_ref, o_ref):
        d = dp_ref[0] + dp_ref[1]
        deg = d[:, 0:1] + 1.0
        o_ref[...] = lax.rsqrt(deg)

    return pl.pallas_call(
        body,
        out_shape=jax.ShapeDtypeStruct((NP, 1), F32),
    )(degp)


def _tc_matmul(h_in, W, dinv, affine=None):
    K = h_in.shape[1]

    def body(h_ref, w_ref, dv_ref, *rest):
        if affine is None:
            o0_ref, o1_ref = rest
            a = h_ref[...]
        else:
            s_ref, t_ref, o0_ref, o1_ref = rest
            a = jnp.maximum(h_ref[...] * s_ref[...] + t_ref[...], 0.0)
        mm = jax.lax.dot_general(a, w_ref[...], (((1,), (0,)), ((), ())),
                                 preferred_element_type=F32,
                                 precision=lax.Precision.HIGHEST)
        hs = mm * dv_ref[...]
        o0_ref[...] = hs[:, :HW]
        o1_ref[...] = hs[:, HW:]

    in_specs = [
        pl.BlockSpec((RB, K), lambda i: (i, 0)),
        pl.BlockSpec((K, HP), lambda i: (0, 0)),
        pl.BlockSpec((RB, 1), lambda i: (i, 0)),
    ]
    args = [h_in, W, dinv]
    if affine is not None:
        in_specs += [pl.BlockSpec((1, HP), lambda i: (0, 0))] * 2
        args += [affine[0], affine[1]]
    return pl.pallas_call(
        body,
        grid=(NRB,),
        in_specs=in_specs,
        out_specs=[pl.BlockSpec((RB, HW), lambda i: (i, 0))] * 2,
        out_shape=[jax.ShapeDtypeStruct((NP, HW), F32)] * 2,
    )(*args)


def _tc_reduce(P, dinv, b, g, be):
    def body(p_ref, dv_ref, b_ref, g_ref, be_ref,
             u_ref, s_ref, t_ref, sum_ref, sq_ref):
        i = pl.program_id(0)
        agg = jnp.concatenate([p_ref[0], p_ref[1]], axis=1)
        u = agg * dv_ref[...] + b_ref[...]
        u_ref[...] = u
        rows = lax.broadcasted_iota(I32, (RB, 1), 0) + i * RB
        um = jnp.where(rows < N, u, 0.0)

        @pl.when(i == 0)
        def _():
            sum_ref[...] = jnp.zeros_like(sum_ref)
            sq_ref[...] = jnp.zeros_like(sq_ref)

        sum_ref[...] += jnp.sum(um, axis=0, keepdims=True)
        sq_ref[...] += jnp.sum(um * um, axis=0, keepdims=True)

        @pl.when(i == NRB - 1)
        def _():
            m = sum_ref[...] / N
            v = sq_ref[...] / N - m * m
            sf = g_ref[...] * lax.rsqrt(v + 1e-5)
            s_ref[...] = sf
            t_ref[...] = be_ref[...] - m * sf

    return pl.pallas_call(
        body,
        grid=(NRB,),
        in_specs=[
            pl.BlockSpec((NC, RB, HW), lambda i: (0, i, 0)),
            pl.BlockSpec((RB, 1), lambda i: (i, 0)),
            pl.BlockSpec((1, HP), lambda i: (0, 0)),
            pl.BlockSpec((1, HP), lambda i: (0, 0)),
            pl.BlockSpec((1, HP), lambda i: (0, 0)),
        ],
        out_specs=[
            pl.BlockSpec((RB, HP), lambda i: (i, 0)),
            pl.BlockSpec((1, HP), lambda i: (0, 0)),
            pl.BlockSpec((1, HP), lambda i: (0, 0)),
        ],
        out_shape=[
            jax.ShapeDtypeStruct((NP, HP), F32),
            jax.ShapeDtypeStruct((1, HP), F32),
            jax.ShapeDtypeStruct((1, HP), F32),
        ],
        scratch_shapes=[pltpu.VMEM((1, HP), F32), pltpu.VMEM((1, HP), F32)],
    )(P, dinv, b, g, be)


def _tc_jk(us, sts, Wjk, bjk):
    def body(u0, u1, u2, u3, s0, t0, s1, t1, s2, t2, s3, t3,
             wjk_ref, bjk_ref, o_ref):
        acc = jnp.broadcast_to(bjk_ref[...], (RB, HP))
        for l, (u_ref, s_ref, t_ref) in enumerate(
                ((u0, s0, t0), (u1, s1, t1), (u2, s2, t2), (u3, s3, t3))):
            a = jnp.maximum(u_ref[...] * s_ref[...] + t_ref[...], 0.0)
            acc = acc + jax.lax.dot_general(
                a, wjk_ref[l], (((1,), (0,)), ((), ())),
                preferred_element_type=F32, precision=lax.Precision.HIGHEST)
        o_ref[...] = acc

    in_specs = [pl.BlockSpec((RB, HP), lambda i: (i, 0))] * 4
    in_specs += [pl.BlockSpec((1, HP), lambda i: (0, 0))] * 8
    in_specs += [pl.BlockSpec((4, HP, HP), lambda i: (0, 0, 0)),
                 pl.BlockSpec((1, HP), lambda i: (0, 0))]
    args = list(us)
    for (s, t) in sts:
        args += [s, t]
    args += [Wjk, bjk]
    return pl.pallas_call(
        body,
        grid=(NRB,),
        in_specs=in_specs,
        out_specs=pl.BlockSpec((RB, HP), lambda i: (i, 0)),
        out_shape=jax.ShapeDtypeStruct((NP, HP), F32),
    )(*args)


def _tc_bounds(batch2):
    NB, BL = batch2.shape

    def body(b_ref, o_ref):
        idx = lax.broadcasted_iota(I32, (NW * LAN, 1), 0)
        th = (idx // LAN) * 8 + idx % LAN
        acc = jnp.zeros((NW * LAN, 1), I32)
        for k in range(NB):
            b = b_ref[pl.ds(k, 1), :]
            acc = acc + jnp.sum(jnp.where(b < th, 1, 0).astype(I32),
                                axis=1, keepdims=True)
        o_ref[...] = acc.reshape(NW, LAN)

    return pl.pallas_call(
        body,
        out_shape=jax.ShapeDtypeStruct((NW, LAN), I32),
    )(batch2)


def _tc_head(pooled, Wlin, blin):
    def body(p_ref, w_ref, b_ref, o_ref):
        o_ref[...] = jax.lax.dot_general(
            p_ref[...], w_ref[...], (((1,), (0,)), ((), ())),
            preferred_element_type=F32,
            precision=lax.Precision.HIGHEST) + b_ref[...]

    return pl.pallas_call(
        body,
        out_shape=jax.ShapeDtypeStruct((G, 1), F32),
    )(pooled, Wlin, blin)


def _pad_rows(a, rows):
    return jnp.pad(a, ((0, rows - a.shape[0]), (0, 0)))


def _pad_feat(v):
    return jnp.pad(v.reshape(1, -1), ((0, 0), (0, HP - v.shape[0])))


def kernel(x, edge_index, batch, W0, b0, g0, be0, W1, b1, g1, be1,
           W2, b2, g2, be2, W3, b3, g3, be3, Wjk, bjk, Wlin, blin):
    src = edge_index[0].astype(I32)
    dst = edge_index[1].astype(I32)
    srcp = jnp.concatenate([src, jnp.zeros((EP - E,), I32)])
    dstp = jnp.concatenate([dst, jnp.full((EP - E,), N, I32)])

    xp = _pad_rows(x, NP)
    W0p = jnp.pad(W0, ((0, 0), (0, HP - H)))
    W1p = jnp.pad(W1, ((0, HP - H), (0, HP - H)))
    W2p = jnp.pad(W2, ((0, HP - H), (0, HP - H)))
    W3p = jnp.pad(W3, ((0, HP - H), (0, HP - H)))
    bs = [_pad_feat(b) for b in (b0, b1, b2, b3)]
    gs = [_pad_feat(g) for g in (g0, g1, g2, g3)]
    bes = [_pad_feat(b) for b in (be0, be1, be2, be3)]
    Wjkp = jnp.pad(Wjk.reshape(4, H, H), ((0, 0), (0, HP - H), (0, HP - H)))
    bjkp = _pad_feat(bjk)
    Wlinp = jnp.pad(Wlin, ((0, HP - H), (0, 0)))
    blinp = blin.reshape(1, 1)

    zeros_rows = jnp.zeros((CH, LAN), F32)
    ones_rows = jnp.ones((CH, LAN), F32)

    degp = _sc_degree(dstp, zeros_rows, ones_rows)
    dinv = _tc_dinv(degp)

    us, sts = [], []
    h_in, W_l, aff = xp, W0p, None
    for l in range(4):
        hs0, hs1 = _tc_matmul(h_in, W_l, dinv, affine=aff)
        P = _sc_aggregate(hs0, hs1, srcp, dstp)
        u, s_l, t_l = _tc_reduce(P, dinv, bs[l], gs[l], bes[l])
        us.append(u)
        sts.append((s_l, t_l))
        if l < 3:
            h_in, W_l, aff = u, (W1p, W2p, W3p)[l], sts[-1]

    hjk = _tc_jk(us, sts, Wjkp, bjkp)
    starts = _tc_bounds(batch.astype(I32).reshape(10, N // 10))
    pooled = _sc_segmax(hjk, starts)
    return _tc_head(pooled, Wlinp, blinp)


# speedup vs baseline: 13.2242x; 1.6751x over previous
import functools

import jax
import jax.numpy as jnp
from jax import lax
from jax.experimental import pallas as pl
from jax.experimental.pallas import tpu as pltpu
from jax.experimental.pallas import tpu_sc as plsc

NC = 2
NS = 16
LAN = 16
NW = NC * NS

N = 10000
E = 320000
G = 256
H = 185
HP = 192
NP = 10112
RB = 1264
NRB = NP // RB
NPS = NP // NS
HW = HP // 2
HQ = HP // 4
CH = 128
CPW = 80
EPW = CH * CPW
EP = EPW * NW
CPS = 160
EPS = CH * CPS
NBUF = 4
F32 = jnp.float32
I32 = jnp.int32

_mesh = functools.partial(plsc.VectorSubcoreMesh, core_axis_name="c",
                          subcore_axis_name="s")
_SC_PARAMS = pltpu.CompilerParams(use_tc_tiling_on_sc=False,
                                  needs_layout_passes=False)


def _sc_degree(dstp, zeros_rows, ones_rows):
    @functools.partial(
        pl.kernel,
        out_type=jax.ShapeDtypeStruct((NC, NP, LAN), F32),
        mesh=_mesh(),
        compiler_params=_SC_PARAMS,
        scratch_types=[
            pltpu.VMEM((CH,), I32),
            pltpu.VMEM((CH, LAN), F32),
            pltpu.VMEM_SHARED((NP, LAN), F32),
        ],
    )
    def k(dst_hbm, zero_hbm, ones_hbm, out_hbm, didx, ones_v, acc):
        c = lax.axis_index("c")
        s = lax.axis_index("s")
        w = s * NC + c
        base = s * NPS
        pltpu.sync_copy(ones_hbm, ones_v)
        for j in range(4):
            pltpu.sync_copy(zero_hbm.at[pl.ds(0, CH)],
                            acc.at[pl.ds(base + j * CH, CH)])
        pltpu.sync_copy(zero_hbm.at[pl.ds(0, NPS - 4 * CH)],
                        acc.at[pl.ds(base + 4 * CH, NPS - 4 * CH)])
        plsc.subcore_barrier()
        woff = w * EPW

        @pl.loop(0, CPW)
        def _(i):
            pltpu.sync_copy(dst_hbm.at[pl.ds(woff + i * CH, CH)], didx)
            pltpu.sync_copy(ones_v, acc.at[didx], add=True)

        plsc.subcore_barrier()
        pltpu.sync_copy(acc.at[pl.ds(base, NPS)], out_hbm.at[c, pl.ds(base, NPS)])

    return k(dstp, zeros_rows, ones_rows)


def _sc_aggregate(hsq, srcp, dstp):
    @functools.partial(
        pl.kernel,
        out_type=jax.ShapeDtypeStruct((2 * NC, NP, HQ), F32),
        mesh=_mesh(),
        compiler_params=_SC_PARAMS,
        scratch_types=[
            pltpu.VMEM((CPS, CH), I32),
            pltpu.VMEM((CPS, CH), I32),
            [pltpu.VMEM((CH, HQ), F32)] * NBUF,
            pltpu.VMEM_SHARED((NP, HQ), F32),
            pltpu.VMEM_SHARED((NP, HQ), F32),
            [pltpu.SemaphoreType.DMA] * NBUF,
        ],
    )
    def k(h0_hbm, h1_hbm, h2_hbm, h3_hbm, src_hbm, dst_hbm, out_hbm,
          sidx, didx, rows, srctab, acc, gsems):
        c = lax.axis_index("c")
        s = lax.axis_index("s")
        base = s * NPS

        pltpu.sync_copy(src_hbm.at[s], sidx)
        pltpu.sync_copy(dst_hbm.at[s], didx)

        def gather(i, b):
            return pltpu.async_copy(srctab.at[sidx.at[i]], rows[b], gsems[b])

        def run(h_hbm, p):
            pltpu.sync_copy(h_hbm.at[pl.ds(base, NPS)],
                            srctab.at[pl.ds(base, NPS)])
            pltpu.sync_copy(h_hbm.at[pl.ds(base, NPS)], acc.at[pl.ds(base, NPS)])
            plsc.subcore_barrier()

            for b in range(NBUF):
                gather(b, b)

            @pl.loop(0, CPS // NBUF)
            def _(kk):
                i0 = kk * NBUF
                for b in range(NBUF):
                    pltpu.make_async_copy(srctab.at[sidx.at[i0 + b]],
                                          rows[b], gsems[b]).wait()
                    pltpu.sync_copy(rows[b], acc.at[didx.at[i0 + b]], add=True)

                    @pl.when(kk < CPS // NBUF - 1)
                    def _():
                        gather(i0 + NBUF + b, b)

            plsc.subcore_barrier()
            pltpu.sync_copy(acc.at[pl.ds(base, NPS)],
                            out_hbm.at[p, pl.ds(base, NPS)])
            plsc.subcore_barrier()

        @pl.when(c == 0)
        def _():
            run(h0_hbm, 0)
            run(h1_hbm, 1)

        @pl.when(c == 1)
        def _():
            run(h2_hbm, 2)
            run(h3_hbm, 3)

    return k(*hsq, srcp, dstp)


def _sc_segmax(hjk, starts):
    NCB = HP // LAN

    @functools.partial(
        pl.kernel,
        out_type=jax.ShapeDtypeStruct((G, HP), F32),
        mesh=_mesh(),
        compiler_params=_SC_PARAMS,
        scratch_types=[
            pltpu.VMEM((LAN,), I32),
            pltpu.VMEM((LAN, HP), F32),
            pltpu.VMEM((1, HP), F32),
        ],
    )
    def k(hjk_hbm, starts_hbm, out_hbm, stv, buf, orow):
        c = lax.axis_index("c")
        s = lax.axis_index("s")
        w = s * NC + c
        pltpu.sync_copy(starts_hbm.at[w], stv)
        lane = lax.iota(I32, LAN)
        sv = stv[...]

        def getscal(j):
            return jnp.max(jnp.where(lane == j, sv, 0))

        for j in range(8):
            glo = getscal(j)
            ghi = getscal(j + 1)
            cbase = jnp.bitwise_and(glo, -LAN)
            nch = (ghi - cbase + (LAN - 1)) // LAN
            neg = jnp.full((LAN,), -jnp.inf, F32)
            acc0 = tuple(neg for _ in range(NCB))

            def chunk(kk, accs, glo=glo, ghi=ghi, cbase=cbase):
                c0 = cbase + kk * LAN
                pltpu.sync_copy(hjk_hbm.at[pl.ds(c0, LAN)], buf)
                out = list(accs)
                for r in range(LAN):
                    valid = jnp.logical_and(c0 + r >= glo, c0 + r < ghi)
                    for cb in range(NCB):
                        v = buf[r, pl.ds(cb * LAN, LAN)]
                        v = jnp.where(valid, v, neg)
                        out[cb] = jnp.maximum(out[cb], v)
                return tuple(out)

            accs = lax.fori_loop(0, nch, chunk, acc0)
            for cb in range(NCB):
                orow[0, pl.ds(cb * LAN, LAN)] = accs[cb]
            g = w * 8 + j
            pltpu.sync_copy(orow, out_hbm.at[pl.ds(g, 1)])

    return k(hjk, starts)


def _tc_dinv(degp):
    def body(dp_ref, o_ref):
        d = dp_ref[0] + dp_ref[1]
        deg = d[:, 0:1] + 1.0
        o_ref[...] = lax.rsqrt(deg)

    return pl.pallas_call(
        body,
        out_shape=jax.ShapeDtypeStruct((NP, 1), F32),
    )(degp)


def _tc_matmul(h_in, W, dinv, affine=None):
    K = h_in.shape[1]

    def body(h_ref, w_ref, dv_ref, *rest):
        if affine is None:
            a = h_ref[...]
            o_refs = rest
        else:
            s_ref, t_ref = rest[:2]
            o_refs = rest[2:]
            a = jnp.maximum(h_ref[...] * s_ref[...] + t_ref[...], 0.0)
        mm = jax.lax.dot_general(a, w_ref[...], (((1,), (0,)), ((), ())),
                                 preferred_element_type=F32,
                                 precision=lax.Precision.HIGHEST)
        hs = mm * dv_ref[...]
        for q in range(4):
            o_refs[q][...] = hs[:, q * HQ:(q + 1) * HQ]

    in_specs = [
        pl.BlockSpec((RB, K), lambda i: (i, 0)),
        pl.BlockSpec((K, HP), lambda i: (0, 0)),
        pl.BlockSpec((RB, 1), lambda i: (i, 0)),
    ]
    args = [h_in, W, dinv]
    if affine is not None:
        in_specs += [pl.BlockSpec((1, HP), lambda i: (0, 0))] * 2
        args += [affine[0], affine[1]]
    return pl.pallas_call(
        body,
        grid=(NRB,),
        in_specs=in_specs,
        out_specs=[pl.BlockSpec((RB, HQ), lambda i: (i, 0))] * 4,
        out_shape=[jax.ShapeDtypeStruct((NP, HQ), F32)] * 4,
    )(*args)


def _tc_reduce(P, dinv, b, g, be):
    def body(p_ref, dv_ref, b_ref, g_ref, be_ref,
             u_ref, s_ref, t_ref, sum_ref, sq_ref):
        i = pl.program_id(0)
        agg = jnp.concatenate([p_ref[0], p_ref[1], p_ref[2], p_ref[3]],
                              axis=1)
        u = agg * dv_ref[...] + b_ref[...]
        u_ref[...] = u
        rows = lax.broadcasted_iota(I32, (RB, 1), 0) + i * RB
        um = jnp.where(rows < N, u, 0.0)

        @pl.when(i == 0)
        def _():
            sum_ref[...] = jnp.zeros_like(sum_ref)
            sq_ref[...] = jnp.zeros_like(sq_ref)

        sum_ref[...] += jnp.sum(um, axis=0, keepdims=True)
        sq_ref[...] += jnp.sum(um * um, axis=0, keepdims=True)

        @pl.when(i == NRB - 1)
        def _():
            m = sum_ref[...] / N
            v = sq_ref[...] / N - m * m
            sf = g_ref[...] * lax.rsqrt(v + 1e-5)
            s_ref[...] = sf
            t_ref[...] = be_ref[...] - m * sf

    return pl.pallas_call(
        body,
        grid=(NRB,),
        in_specs=[
            pl.BlockSpec((2 * NC, RB, HQ), lambda i: (0, i, 0)),
            pl.BlockSpec((RB, 1), lambda i: (i, 0)),
            pl.BlockSpec((1, HP), lambda i: (0, 0)),
            pl.BlockSpec((1, HP), lambda i: (0, 0)),
            pl.BlockSpec((1, HP), lambda i: (0, 0)),
        ],
        out_specs=[
            pl.BlockSpec((RB, HP), lambda i: (i, 0)),
            pl.BlockSpec((1, HP), lambda i: (0, 0)),
            pl.BlockSpec((1, HP), lambda i: (0, 0)),
        ],
        out_shape=[
            jax.ShapeDtypeStruct((NP, HP), F32),
            jax.ShapeDtypeStruct((1, HP), F32),
            jax.ShapeDtypeStruct((1, HP), F32),
        ],
        scratch_shapes=[pltpu.VMEM((1, HP), F32), pltpu.VMEM((1, HP), F32)],
    )(P, dinv, b, g, be)


def _tc_jk(us, sts, Wjk, bjk):
    def body(u0, u1, u2, u3, s0, t0, s1, t1, s2, t2, s3, t3,
             wjk_ref, bjk_ref, o_ref):
        acc = jnp.broadcast_to(bjk_ref[...], (RB, HP))
        for l, (u_ref, s_ref, t_ref) in enumerate(
                ((u0, s0, t0), (u1, s1, t1), (u2, s2, t2), (u3, s3, t3))):
            a = jnp.maximum(u_ref[...] * s_ref[...] + t_ref[...], 0.0)
            acc = acc + jax.lax.dot_general(
                a, wjk_ref[l], (((1,), (0,)), ((), ())),
                preferred_element_type=F32, precision=lax.Precision.HIGHEST)
        o_ref[...] = acc

    in_specs = [pl.BlockSpec((RB, HP), lambda i: (i, 0))] * 4
    in_specs += [pl.BlockSpec((1, HP), lambda i: (0, 0))] * 8
    in_specs += [pl.BlockSpec((4, HP, HP), lambda i: (0, 0, 0)),
                 pl.BlockSpec((1, HP), lambda i: (0, 0))]
    args = list(us)
    for (s, t) in sts:
        args += [s, t]
    args += [Wjk, bjk]
    return pl.pallas_call(
        body,
        grid=(NRB,),
        in_specs=in_specs,
        out_specs=pl.BlockSpec((RB, HP), lambda i: (i, 0)),
        out_shape=jax.ShapeDtypeStruct((NP, HP), F32),
    )(*args)


def _tc_bounds(batch2):
    NB, BL = batch2.shape

    def body(b_ref, o_ref):
        idx = lax.broadcasted_iota(I32, (NW * LAN, 1), 0)
        th = (idx // LAN) * 8 + idx % LAN
        acc = jnp.zeros((NW * LAN, 1), I32)
        for k in range(NB):
            b = b_ref[pl.ds(k, 1), :]
            acc = acc + jnp.sum(jnp.where(b < th, 1, 0).astype(I32),
                                axis=1, keepdims=True)
        o_ref[...] = acc.reshape(NW, LAN)

    return pl.pallas_call(
        body,
        out_shape=jax.ShapeDtypeStruct((NW, LAN), I32),
    )(batch2)


def _tc_head(pooled, Wlin, blin):
    def body(p_ref, w_ref, b_ref, o_ref):
        o_ref[...] = jax.lax.dot_general(
            p_ref[...], w_ref[...], (((1,), (0,)), ((), ())),
            preferred_element_type=F32,
            precision=lax.Precision.HIGHEST) + b_ref[...]

    return pl.pallas_call(
        body,
        out_shape=jax.ShapeDtypeStruct((G, 1), F32),
    )(pooled, Wlin, blin)


def _pad_rows(a, rows):
    return jnp.pad(a, ((0, rows - a.shape[0]), (0, 0)))


def _pad_feat(v):
    return jnp.pad(v.reshape(1, -1), ((0, 0), (0, HP - v.shape[0])))


def kernel(x, edge_index, batch, W0, b0, g0, be0, W1, b1, g1, be1,
           W2, b2, g2, be2, W3, b3, g3, be3, Wjk, bjk, Wlin, blin):
    src = edge_index[0].astype(I32)
    dst = edge_index[1].astype(I32)
    srcp = jnp.concatenate([src, jnp.zeros((EP - E,), I32)])
    dstp = jnp.concatenate([dst, jnp.full((EP - E,), N, I32)])
    srcp3 = srcp.reshape(NS, CPS, CH)
    dstp3 = dstp.reshape(NS, CPS, CH)

    xp = _pad_rows(x, NP)
    W0p = jnp.pad(W0, ((0, 0), (0, HP - H)))
    W1p = jnp.pad(W1, ((0, HP - H), (0, HP - H)))
    W2p = jnp.pad(W2, ((0, HP - H), (0, HP - H)))
    W3p = jnp.pad(W3, ((0, HP - H), (0, HP - H)))
    bs = [_pad_feat(b) for b in (b0, b1, b2, b3)]
    gs = [_pad_feat(g) for g in (g0, g1, g2, g3)]
    bes = [_pad_feat(b) for b in (be0, be1, be2, be3)]
    Wjkp = jnp.pad(Wjk.reshape(4, H, H), ((0, 0), (0, HP - H), (0, HP - H)))
    bjkp = _pad_feat(bjk)
    Wlinp = jnp.pad(Wlin, ((0, HP - H), (0, 0)))
    blinp = blin.reshape(1, 1)

    zeros_rows = jnp.zeros((CH, LAN), F32)
    ones_rows = jnp.ones((CH, LAN), F32)

    degp = _sc_degree(dstp, zeros_rows, ones_rows)
    dinv = _tc_dinv(degp)

    us, sts = [], []
    h_in, W_l, aff = xp, W0p, None
    for l in range(4):
        hsq = _tc_matmul(h_in, W_l, dinv, affine=aff)
        P = _sc_aggregate(hsq, srcp3, dstp3)
        u, s_l, t_l = _tc_reduce(P, dinv, bs[l], gs[l], bes[l])
        us.append(u)
        sts.append((s_l, t_l))
        if l < 3:
            h_in, W_l, aff = u, (W1p, W2p, W3p)[l], sts[-1]

    hjk = _tc_jk(us, sts, Wjkp, bjkp)
    starts = _tc_bounds(batch.astype(I32).reshape(10, N // 10))
    pooled = _sc_segmax(hjk, starts)
    return _tc_head(pooled, Wlinp, blinp)


# speedup vs baseline: 14.6236x; 1.1058x over previous
import functools

import jax
import jax.numpy as jnp
from jax import lax
from jax.experimental import pallas as pl
from jax.experimental.pallas import tpu as pltpu
from jax.experimental.pallas import tpu_sc as plsc

NC = 2
NS = 16
LAN = 16
NW = NC * NS

N = 10000
E = 320000
G = 256
H = 185
HP = 192
NP = 10112
RB = 1264
NRB = NP // RB
NPS = NP // NS
HW = HP // 2
HQ = HP // 4
CH = 128
CPW = 80
EPW = CH * CPW
EP = EPW * NW
CPS = 160
EPS = CH * CPS
NBUF = 4
F32 = jnp.float32
I32 = jnp.int32

_mesh = functools.partial(plsc.VectorSubcoreMesh, core_axis_name="c",
                          subcore_axis_name="s")
_SC_PARAMS = pltpu.CompilerParams(use_tc_tiling_on_sc=False,
                                  needs_layout_passes=False)


def _sc_degree(dstp, zeros_rows, ones_rows):
    @functools.partial(
        pl.kernel,
        out_type=jax.ShapeDtypeStruct((NC, NP, LAN), F32),
        mesh=_mesh(),
        compiler_params=_SC_PARAMS,
        scratch_types=[
            pltpu.VMEM((CH,), I32),
            pltpu.VMEM((CH, LAN), F32),
            pltpu.VMEM_SHARED((NP, LAN), F32),
        ],
    )
    def k(dst_hbm, zero_hbm, ones_hbm, out_hbm, didx, ones_v, acc):
        c = lax.axis_index("c")
        s = lax.axis_index("s")
        w = s * NC + c
        base = s * NPS
        pltpu.sync_copy(ones_hbm, ones_v)
        for j in range(4):
            pltpu.sync_copy(zero_hbm.at[pl.ds(0, CH)],
                            acc.at[pl.ds(base + j * CH, CH)])
        pltpu.sync_copy(zero_hbm.at[pl.ds(0, NPS - 4 * CH)],
                        acc.at[pl.ds(base + 4 * CH, NPS - 4 * CH)])
        plsc.subcore_barrier()
        woff = w * EPW

        @pl.loop(0, CPW)
        def _(i):
            pltpu.sync_copy(dst_hbm.at[pl.ds(woff + i * CH, CH)], didx)
            pltpu.sync_copy(ones_v, acc.at[didx], add=True)

        plsc.subcore_barrier()
        pltpu.sync_copy(acc.at[pl.ds(base, NPS)], out_hbm.at[c, pl.ds(base, NPS)])

    return k(dstp, zeros_rows, ones_rows)


def _sc_aggregate(hsq, srcp, dstp):
    @functools.partial(
        pl.kernel,
        out_type=jax.ShapeDtypeStruct((2 * NC, NP, HQ), F32),
        mesh=_mesh(),
        compiler_params=_SC_PARAMS,
        scratch_types=[
            pltpu.VMEM((CPS, CH), I32),
            pltpu.VMEM((CPS, CH), I32),
            [pltpu.VMEM((CH, HQ), F32)] * NBUF,
            pltpu.VMEM_SHARED((NP, HQ), F32),
            pltpu.VMEM_SHARED((NP, HQ), F32),
            [pltpu.SemaphoreType.DMA] * NBUF,
            [pltpu.SemaphoreType.DMA] * NBUF,
        ],
    )
    def k(h0_hbm, h1_hbm, h2_hbm, h3_hbm, src_hbm, dst_hbm, out_hbm,
          sidx, didx, rows, srctab, acc, gsems, ssems):
        c = lax.axis_index("c")
        s = lax.axis_index("s")
        base = s * NPS

        pltpu.sync_copy(src_hbm.at[s], sidx)
        pltpu.sync_copy(dst_hbm.at[s], didx)

        def gather(i, b):
            return pltpu.async_copy(srctab.at[sidx.at[i]], rows[b], gsems[b])

        def run(h_hbm, p):
            pltpu.sync_copy(h_hbm.at[pl.ds(base, NPS)],
                            srctab.at[pl.ds(base, NPS)])
            pltpu.sync_copy(h_hbm.at[pl.ds(base, NPS)], acc.at[pl.ds(base, NPS)])
            plsc.subcore_barrier()

            gather(0, 0)
            gather(1, 1)

            @pl.loop(0, CPS // NBUF)
            def _(kk):
                j0 = kk * NBUF
                for b in range(NBUF):
                    j = j0 + b
                    br = (b + 2) % NBUF
                    pltpu.make_async_copy(srctab.at[sidx.at[j]],
                                          rows[b], gsems[b]).wait()
                    pltpu.async_copy(rows[b], acc.at[didx.at[j]], ssems[b],
                                     add=True)

                    @pl.when(j >= 2)
                    def _():
                        pltpu.make_async_copy(
                            rows[br], acc.at[didx.at[j]], ssems[br]).wait()

                    @pl.when(j + 2 < CPS)
                    def _():
                        gather(j + 2, br)

            for jd in (CPS - 2, CPS - 1):
                bd = jd % NBUF
                pltpu.make_async_copy(rows[bd], acc.at[didx.at[jd]],
                                      ssems[bd]).wait()
            plsc.subcore_barrier()
            pltpu.sync_copy(acc.at[pl.ds(base, NPS)],
                            out_hbm.at[p, pl.ds(base, NPS)])
            plsc.subcore_barrier()

        @pl.when(c == 0)
        def _():
            run(h0_hbm, 0)
            run(h1_hbm, 1)

        @pl.when(c == 1)
        def _():
            run(h2_hbm, 2)
            run(h3_hbm, 3)

    return k(*hsq, srcp, dstp)


def _sc_segmax(hjk, starts):
    NCB = HP // LAN

    @functools.partial(
        pl.kernel,
        out_type=jax.ShapeDtypeStruct((G, HP), F32),
        mesh=_mesh(),
        compiler_params=_SC_PARAMS,
        scratch_types=[
            pltpu.VMEM((LAN,), I32),
            pltpu.VMEM((LAN, HP), F32),
            pltpu.VMEM((1, HP), F32),
        ],
    )
    def k(hjk_hbm, starts_hbm, out_hbm, stv, buf, orow):
        c = lax.axis_index("c")
        s = lax.axis_index("s")
        w = s * NC + c
        pltpu.sync_copy(starts_hbm.at[w], stv)
        lane = lax.iota(I32, LAN)
        sv = stv[...]

        def getscal(j):
            return jnp.max(jnp.where(lane == j, sv, 0))

        for j in range(8):
            glo = getscal(j)
            ghi = getscal(j + 1)
            cbase = jnp.bitwise_and(glo, -LAN)
            nch = (ghi - cbase + (LAN - 1)) // LAN
            neg = jnp.full((LAN,), -jnp.inf, F32)
            acc0 = tuple(neg for _ in range(NCB))

            def chunk(kk, accs, glo=glo, ghi=ghi, cbase=cbase):
                c0 = cbase + kk * LAN
                pltpu.sync_copy(hjk_hbm.at[pl.ds(c0, LAN)], buf)
                out = list(accs)
                for r in range(LAN):
                    valid = jnp.logical_and(c0 + r >= glo, c0 + r < ghi)
                    for cb in range(NCB):
                        v = buf[r, pl.ds(cb * LAN, LAN)]
                        v = jnp.where(valid, v, neg)
                        out[cb] = jnp.maximum(out[cb], v)
                return tuple(out)

            accs = lax.fori_loop(0, nch, chunk, acc0)
            for cb in range(NCB):
                orow[0, pl.ds(cb * LAN, LAN)] = accs[cb]
            g = w * 8 + j
            pltpu.sync_copy(orow, out_hbm.at[pl.ds(g, 1)])

    return k(hjk, starts)


def _tc_dinv(degp, m0q):
    def body(dp_ref, m0, m1, m2, m3, o_ref, h0, h1, h2, h3):
        d = dp_ref[0] + dp_ref[1]
        deg = d[:, 0:1] + 1.0
        dinv = lax.rsqrt(deg)
        o_ref[...] = dinv
        for m_ref, h_ref in ((m0, h0), (m1, h1), (m2, h2), (m3, h3)):
            h_ref[...] = m_ref[...] * dinv

    return pl.pallas_call(
        body,
        grid=(NRB,),
        in_specs=[pl.BlockSpec((NC, RB, LAN), lambda i: (0, i, 0))]
        + [pl.BlockSpec((RB, HQ), lambda i: (i, 0))] * 4,
        out_specs=[pl.BlockSpec((RB, 1), lambda i: (i, 0))]
        + [pl.BlockSpec((RB, HQ), lambda i: (i, 0))] * 4,
        out_shape=[jax.ShapeDtypeStruct((NP, 1), F32)]
        + [jax.ShapeDtypeStruct((NP, HQ), F32)] * 4,
    )(degp, *m0q)


def _tc_matmul(h_in, W, dinv=None, affine=None):
    K = h_in.shape[1]

    def body(h_ref, w_ref, *rest):
        if dinv is not None:
            dv_ref, rest = rest[0], rest[1:]
        if affine is None:
            a = h_ref[...]
            o_refs = rest
        else:
            s_ref, t_ref = rest[:2]
            o_refs = rest[2:]
            a = jnp.maximum(h_ref[...] * s_ref[...] + t_ref[...], 0.0)
        mm = jax.lax.dot_general(a, w_ref[...], (((1,), (0,)), ((), ())),
                                 preferred_element_type=F32,
                                 precision=lax.Precision.HIGHEST)
        hs = mm * dv_ref[...] if dinv is not None else mm
        for q in range(4):
            o_refs[q][...] = hs[:, q * HQ:(q + 1) * HQ]

    in_specs = [
        pl.BlockSpec((RB, K), lambda i: (i, 0)),
        pl.BlockSpec((K, HP), lambda i: (0, 0)),
    ]
    args = [h_in, W]
    if dinv is not None:
        in_specs += [pl.BlockSpec((RB, 1), lambda i: (i, 0))]
        args += [dinv]
    if affine is not None:
        in_specs += [pl.BlockSpec((1, HP), lambda i: (0, 0))] * 2
        args += [affine[0], affine[1]]
    return pl.pallas_call(
        body,
        grid=(NRB,),
        in_specs=in_specs,
        out_specs=[pl.BlockSpec((RB, HQ), lambda i: (i, 0))] * 4,
        out_shape=[jax.ShapeDtypeStruct((NP, HQ), F32)] * 4,
    )(*args)


def _tc_reduce(P, dinv, b, g, be):
    def body(p_ref, dv_ref, b_ref, g_ref, be_ref,
             u_ref, s_ref, t_ref, sum_ref, sq_ref):
        i = pl.program_id(0)
        agg = jnp.concatenate([p_ref[0], p_ref[1], p_ref[2], p_ref[3]],
                              axis=1)
        u = agg * dv_ref[...] + b_ref[...]
        u_ref[...] = u
        rows = lax.broadcasted_iota(I32, (RB, 1), 0) + i * RB
        um = jnp.where(rows < N, u, 0.0)

        @pl.when(i == 0)
        def _():
            sum_ref[...] = jnp.zeros_like(sum_ref)
            sq_ref[...] = jnp.zeros_like(sq_ref)

        sum_ref[...] += jnp.sum(um, axis=0, keepdims=True)
        sq_ref[...] += jnp.sum(um * um, axis=0, keepdims=True)

        @pl.when(i == NRB - 1)
        def _():
            m = sum_ref[...] / N
            v = sq_ref[...] / N - m * m
            sf = g_ref[...] * lax.rsqrt(v + 1e-5)
            s_ref[...] = sf
            t_ref[...] = be_ref[...] - m * sf

    return pl.pallas_call(
        body,
        grid=(NRB,),
        in_specs=[
            pl.BlockSpec((2 * NC, RB, HQ), lambda i: (0, i, 0)),
            pl.BlockSpec((RB, 1), lambda i: (i, 0)),
            pl.BlockSpec((1, HP), lambda i: (0, 0)),
            pl.BlockSpec((1, HP), lambda i: (0, 0)),
            pl.BlockSpec((1, HP), lambda i: (0, 0)),
        ],
        out_specs=[
            pl.BlockSpec((RB, HP), lambda i: (i, 0)),
            pl.BlockSpec((1, HP), lambda i: (0, 0)),
            pl.BlockSpec((1, HP), lambda i: (0, 0)),
        ],
        out_shape=[
            jax.ShapeDtypeStruct((NP, HP), F32),
            jax.ShapeDtypeStruct((1, HP), F32),
            jax.ShapeDtypeStruct((1, HP), F32),
        ],
        scratch_shapes=[pltpu.VMEM((1, HP), F32), pltpu.VMEM((1, HP), F32)],
    )(P, dinv, b, g, be)


def _tc_jk(us, sts, Wjk, bjk):
    def body(u0, u1, u2, u3, s0, t0, s1, t1, s2, t2, s3, t3,
             wjk_ref, bjk_ref, o_ref):
        acc = jnp.broadcast_to(bjk_ref[...], (RB, HP))
        for l, (u_ref, s_ref, t_ref) in enumerate(
                ((u0, s0, t0), (u1, s1, t1), (u2, s2, t2), (u3, s3, t3))):
            a = jnp.maximum(u_ref[...] * s_ref[...] + t_ref[...], 0.0)
            acc = acc + jax.lax.dot_general(
                a, wjk_ref[l], (((1,), (0,)), ((), ())),
                preferred_element_type=F32, precision=lax.Precision.HIGHEST)
        o_ref[...] = acc

    in_specs = [pl.BlockSpec((RB, HP), lambda i: (i, 0))] * 4
    in_specs += [pl.BlockSpec((1, HP), lambda i: (0, 0))] * 8
    in_specs += [pl.BlockSpec((4, HP, HP), lambda i: (0, 0, 0)),
                 pl.BlockSpec((1, HP), lambda i: (0, 0))]
    args = list(us)
    for (s, t) in sts:
        args += [s, t]
    args += [Wjk, bjk]
    return pl.pallas_call(
        body,
        grid=(NRB,),
        in_specs=in_specs,
        out_specs=pl.BlockSpec((RB, HP), lambda i: (i, 0)),
        out_shape=jax.ShapeDtypeStruct((NP, HP), F32),
    )(*args)


def _tc_bounds(batch2):
    NB, BL = batch2.shape

    def body(b_ref, o_ref):
        idx = lax.broadcasted_iota(I32, (NW * LAN, 1), 0)
        th = (idx // LAN) * 8 + idx % LAN
        acc = jnp.zeros((NW * LAN, 1), I32)
        for k in range(NB):
            b = b_ref[pl.ds(k, 1), :]
            acc = acc + jnp.sum(jnp.where(b < th, 1, 0).astype(I32),
                                axis=1, keepdims=True)
        o_ref[...] = acc.reshape(NW, LAN)

    return pl.pallas_call(
        body,
        out_shape=jax.ShapeDtypeStruct((NW, LAN), I32),
    )(batch2)


def _tc_head(pooled, Wlin, blin):
    def body(p_ref, w_ref, b_ref, o_ref):
        o_ref[...] = jax.lax.dot_general(
            p_ref[...], w_ref[...], (((1,), (0,)), ((), ())),
            preferred_element_type=F32,
            precision=lax.Precision.HIGHEST) + b_ref[...]

    return pl.pallas_call(
        body,
        out_shape=jax.ShapeDtypeStruct((G, 1), F32),
    )(pooled, Wlin, blin)


def _pad_rows(a, rows):
    return jnp.pad(a, ((0, rows - a.shape[0]), (0, 0)))


def _pad_feat(v):
    return jnp.pad(v.reshape(1, -1), ((0, 0), (0, HP - v.shape[0])))


def kernel(x, edge_index, batch, W0, b0, g0, be0, W1, b1, g1, be1,
           W2, b2, g2, be2, W3, b3, g3, be3, Wjk, bjk, Wlin, blin):
    src = edge_index[0].astype(I32)
    dst = edge_index[1].astype(I32)
    srcp = jnp.concatenate([src, jnp.zeros((EP - E,), I32)])
    dstp = jnp.concatenate([dst, jnp.full((EP - E,), N, I32)])
    srcp3 = srcp.reshape(NS, CPS, CH)
    dstp3 = dstp.reshape(NS, CPS, CH)

    xp = _pad_rows(x, NP)
    W0p = jnp.pad(W0, ((0, 0), (0, HP - H)))
    W1p = jnp.pad(W1, ((0, HP - H), (0, HP - H)))
    W2p = jnp.pad(W2, ((0, HP - H), (0, HP - H)))
    W3p = jnp.pad(W3, ((0, HP - H), (0, HP - H)))
    bs = [_pad_feat(b) for b in (b0, b1, b2, b3)]
    gs = [_pad_feat(g) for g in (g0, g1, g2, g3)]
    bes = [_pad_feat(b) for b in (be0, be1, be2, be3)]
    Wjkp = jnp.pad(Wjk.reshape(4, H, H), ((0, 0), (0, HP - H), (0, HP - H)))
    bjkp = _pad_feat(bjk)
    Wlinp = jnp.pad(Wlin, ((0, HP - H), (0, 0)))
    blinp = blin.reshape(1, 1)

    zeros_rows = jnp.zeros((CH, LAN), F32)
    ones_rows = jnp.ones((CH, LAN), F32)

    degp = _sc_degree(dstp, zeros_rows, ones_rows)
    m0q = _tc_matmul(xp, W0p)
    dinv, *hsq0 = _tc_dinv(degp, m0q)

    us, sts = [], []
    h_in, W_l, aff = None, None, None
    for l in range(4):
        if l == 0:
            hsq = hsq0
        else:
            hsq = _tc_matmul(h_in, W_l, dinv, affine=aff)
        P = _sc_aggregate(hsq, srcp3, dstp3)
        u, s_l, t_l = _tc_reduce(P, dinv, bs[l], gs[l], bes[l])
        us.append(u)
        sts.append((s_l, t_l))
        if l < 3:
            h_in, W_l, aff = u, (W1p, W2p, W3p)[l], sts[-1]

    hjk = _tc_jk(us, sts, Wjkp, bjkp)
    starts = _tc_bounds(batch.astype(I32).reshape(10, N // 10))
    pooled = _sc_segmax(hjk, starts)
    return _tc_head(pooled, Wlinp, blinp)


# speedup vs baseline: 15.0371x; 1.0283x over previous
import functools

import jax
import jax.numpy as jnp
from jax import lax
from jax.experimental import pallas as pl
from jax.experimental.pallas import tpu as pltpu
from jax.experimental.pallas import tpu_sc as plsc

NC = 2
NS = 16
LAN = 16
NW = NC * NS

N = 10000
E = 320000
G = 256
H = 185
HP = 192
NP = 10112
RB = 1264
NRB = NP // RB
NPS = NP // NS
HW = HP // 2
HQ = HP // 4
CH = 128
CPW = 80
EPW = CH * CPW
EP = EPW * NW
CPS = 160
EPS = CH * CPS
NBUF = 4
F32 = jnp.float32
I32 = jnp.int32

_mesh = functools.partial(plsc.VectorSubcoreMesh, core_axis_name="c",
                          subcore_axis_name="s")
_SC_PARAMS = pltpu.CompilerParams(use_tc_tiling_on_sc=False,
                                  needs_layout_passes=False)


def _sc_degree(dstp, zeros_rows, ones_rows):
    @functools.partial(
        pl.kernel,
        out_type=jax.ShapeDtypeStruct((NC, NP, LAN), F32),
        mesh=_mesh(),
        compiler_params=_SC_PARAMS,
        scratch_types=[
            pltpu.VMEM((CH,), I32),
            pltpu.VMEM((CH, LAN), F32),
            pltpu.VMEM_SHARED((NP, LAN), F32),
        ],
    )
    def k(dst_hbm, zero_hbm, ones_hbm, out_hbm, didx, ones_v, acc):
        c = lax.axis_index("c")
        s = lax.axis_index("s")
        w = s * NC + c
        base = s * NPS
        pltpu.sync_copy(ones_hbm, ones_v)
        for j in range(4):
            pltpu.sync_copy(zero_hbm.at[pl.ds(0, CH)],
                            acc.at[pl.ds(base + j * CH, CH)])
        pltpu.sync_copy(zero_hbm.at[pl.ds(0, NPS - 4 * CH)],
                        acc.at[pl.ds(base + 4 * CH, NPS - 4 * CH)])
        plsc.subcore_barrier()
        woff = w * EPW

        @pl.loop(0, CPW)
        def _(i):
            pltpu.sync_copy(dst_hbm.at[pl.ds(woff + i * CH, CH)], didx)
            pltpu.sync_copy(ones_v, acc.at[didx], add=True)

        plsc.subcore_barrier()
        pltpu.sync_copy(acc.at[pl.ds(base, NPS)], out_hbm.at[c, pl.ds(base, NPS)])

    return k(dstp, zeros_rows, ones_rows)


def _sc_aggregate(hsq, srcp, dstp):
    @functools.partial(
        pl.kernel,
        out_type=jax.ShapeDtypeStruct((2 * NC, NP, HQ), F32),
        mesh=_mesh(),
        compiler_params=_SC_PARAMS,
        scratch_types=[
            pltpu.VMEM((CPS, CH), I32),
            pltpu.VMEM((CPS, CH), I32),
            [pltpu.VMEM((CH, HQ), F32)] * NBUF,
            pltpu.VMEM_SHARED((NP, HQ), F32),
            pltpu.VMEM_SHARED((NP, HQ), F32),
            [pltpu.SemaphoreType.DMA] * NBUF,
            [pltpu.SemaphoreType.DMA] * NBUF,
        ],
    )
    def k(h0_hbm, h1_hbm, h2_hbm, h3_hbm, src_hbm, dst_hbm, out_hbm,
          sidx, didx, rows, srctab, acc, gsems, ssems):
        c = lax.axis_index("c")
        s = lax.axis_index("s")
        base = s * NPS

        pltpu.sync_copy(src_hbm.at[s], sidx)
        pltpu.sync_copy(dst_hbm.at[s], didx)

        def gather(i, b):
            return pltpu.async_copy(srctab.at[sidx.at[i]], rows[b], gsems[b])

        def run(h_hbm, p):
            pltpu.sync_copy(h_hbm.at[pl.ds(base, NPS)],
                            srctab.at[pl.ds(base, NPS)])
            pltpu.sync_copy(h_hbm.at[pl.ds(base, NPS)], acc.at[pl.ds(base, NPS)])
            plsc.subcore_barrier()

            gather(0, 0)
            gather(1, 1)

            def slot(j, b, drain=True, refire=True):
                br = (b + 2) % NBUF
                pltpu.make_async_copy(srctab.at[sidx.at[j]],
                                      rows[b], gsems[b]).wait()
                pltpu.async_copy(rows[b], acc.at[didx.at[j]], ssems[b],
                                 add=True)
                if drain:
                    pltpu.make_async_copy(
                        rows[br], acc.at[didx.at[j]], ssems[br]).wait()
                if refire:
                    gather(j + 2, br)

            slot(0, 0, drain=False)
            slot(1, 1, drain=False)
            slot(2, 2)
            slot(3, 3)

            @pl.loop(1, CPS // NBUF - 1)
            def _(kk):
                j0 = kk * NBUF
                for b in range(NBUF):
                    slot(j0 + b, b)

            j0 = CPS - NBUF
            slot(j0, 0)
            slot(j0 + 1, 1)
            slot(j0 + 2, 2, refire=False)
            slot(j0 + 3, 3, refire=False)
            for jd in (CPS - 2, CPS - 1):
                bd = jd % NBUF
                pltpu.make_async_copy(rows[bd], acc.at[didx.at[jd]],
                                      ssems[bd]).wait()
            plsc.subcore_barrier()
            pltpu.sync_copy(acc.at[pl.ds(base, NPS)],
                            out_hbm.at[p, pl.ds(base, NPS)])
            plsc.subcore_barrier()

        @pl.when(c == 0)
        def _():
            run(h0_hbm, 0)
            run(h1_hbm, 1)

        @pl.when(c == 1)
        def _():
            run(h2_hbm, 2)
            run(h3_hbm, 3)

    return k(*hsq, srcp, dstp)


def _sc_segmax(hjk, starts):
    NCB = HP // LAN

    @functools.partial(
        pl.kernel,
        out_type=jax.ShapeDtypeStruct((G, HP), F32),
        mesh=_mesh(),
        compiler_params=_SC_PARAMS,
        scratch_types=[
            pltpu.VMEM((LAN,), I32),
            pltpu.VMEM((LAN, HP), F32),
            pltpu.VMEM((1, HP), F32),
        ],
    )
    def k(hjk_hbm, starts_hbm, out_hbm, stv, buf, orow):
        c = lax.axis_index("c")
        s = lax.axis_index("s")
        w = s * NC + c
        pltpu.sync_copy(starts_hbm.at[w], stv)
        lane = lax.iota(I32, LAN)
        sv = stv[...]

        def getscal(j):
            return jnp.max(jnp.where(lane == j, sv, 0))

        for j in range(8):
            glo = getscal(j)
            ghi = getscal(j + 1)
            cbase = jnp.bitwise_and(glo, -LAN)
            nch = (ghi - cbase + (LAN - 1)) // LAN
            neg = jnp.full((LAN,), -jnp.inf, F32)
            acc0 = tuple(neg for _ in range(NCB))

            def chunk(kk, accs, glo=glo, ghi=ghi, cbase=cbase):
                c0 = cbase + kk * LAN
                pltpu.sync_copy(hjk_hbm.at[pl.ds(c0, LAN)], buf)
                out = list(accs)
                for r in range(LAN):
                    valid = jnp.logical_and(c0 + r >= glo, c0 + r < ghi)
                    for cb in range(NCB):
                        v = buf[r, pl.ds(cb * LAN, LAN)]
                        v = jnp.where(valid, v, neg)
                        out[cb] = jnp.maximum(out[cb], v)
                return tuple(out)

            accs = lax.fori_loop(0, nch, chunk, acc0)
            for cb in range(NCB):
                orow[0, pl.ds(cb * LAN, LAN)] = accs[cb]
            g = w * 8 + j
            pltpu.sync_copy(orow, out_hbm.at[pl.ds(g, 1)])

    return k(hjk, starts)


def _tc_dinv(degp, m0q):
    def body(dp_ref, m0, m1, m2, m3, o_ref, h0, h1, h2, h3):
        d = dp_ref[0] + dp_ref[1]
        deg = d[:, 0:1] + 1.0
        dinv = lax.rsqrt(deg)
        o_ref[...] = dinv
        for m_ref, h_ref in ((m0, h0), (m1, h1), (m2, h2), (m3, h3)):
            h_ref[...] = m_ref[...] * dinv

    return pl.pallas_call(
        body,
        grid=(NRB,),
        in_specs=[pl.BlockSpec((NC, RB, LAN), lambda i: (0, i, 0))]
        + [pl.BlockSpec((RB, HQ), lambda i: (i, 0))] * 4,
        out_specs=[pl.BlockSpec((RB, 1), lambda i: (i, 0))]
        + [pl.BlockSpec((RB, HQ), lambda i: (i, 0))] * 4,
        out_shape=[jax.ShapeDtypeStruct((NP, 1), F32)]
        + [jax.ShapeDtypeStruct((NP, HQ), F32)] * 4,
    )(degp, *m0q)


def _tc_matmul(h_in, W, dinv=None, affine=None):
    K = h_in.shape[1]

    def body(h_ref, w_ref, *rest):
        if dinv is not None:
            dv_ref, rest = rest[0], rest[1:]
        if affine is None:
            a = h_ref[...]
            o_refs = rest
        else:
            s_ref, t_ref = rest[:2]
            o_refs = rest[2:]
            a = jnp.maximum(h_ref[...] * s_ref[...] + t_ref[...], 0.0)
        mm = jax.lax.dot_general(a, w_ref[...], (((1,), (0,)), ((), ())),
                                 preferred_element_type=F32)
        hs = mm * dv_ref[...] if dinv is not None else mm
        for q in range(4):
            o_refs[q][...] = hs[:, q * HQ:(q + 1) * HQ]

    in_specs = [
        pl.BlockSpec((RB, K), lambda i: (i, 0)),
        pl.BlockSpec((K, HP), lambda i: (0, 0)),
    ]
    args = [h_in, W]
    if dinv is not None:
        in_specs += [pl.BlockSpec((RB, 1), lambda i: (i, 0))]
        args += [dinv]
    if affine is not None:
        in_specs += [pl.BlockSpec((1, HP), lambda i: (0, 0))] * 2
        args += [affine[0], affine[1]]
    return pl.pallas_call(
        body,
        grid=(NRB,),
        in_specs=in_specs,
        out_specs=[pl.BlockSpec((RB, HQ), lambda i: (i, 0))] * 4,
        out_shape=[jax.ShapeDtypeStruct((NP, HQ), F32)] * 4,
    )(*args)


def _tc_reduce(P, dinv, b, g, be):
    def body(p_ref, dv_ref, b_ref, g_ref, be_ref,
             u_ref, s_ref, t_ref, sum_ref, sq_ref):
        i = pl.program_id(0)
        agg = jnp.concatenate([p_ref[0], p_ref[1], p_ref[2], p_ref[3]],
                              axis=1)
        u = agg * dv_ref[...] + b_ref[...]
        u_ref[...] = u
        rows = lax.broadcasted_iota(I32, (RB, 1), 0) + i * RB
        um = jnp.where(rows < N, u, 0.0)

        @pl.when(i == 0)
        def _():
            sum_ref[...] = jnp.zeros_like(sum_ref)
            sq_ref[...] = jnp.zeros_like(sq_ref)

        sum_ref[...] += jnp.sum(um, axis=0, keepdims=True)
        sq_ref[...] += jnp.sum(um * um, axis=0, keepdims=True)

        @pl.when(i == NRB - 1)
        def _():
            m = sum_ref[...] / N
            v = sq_ref[...] / N - m * m
            sf = g_ref[...] * lax.rsqrt(v + 1e-5)
            s_ref[...] = sf
            t_ref[...] = be_ref[...] - m * sf

    return pl.pallas_call(
        body,
        grid=(NRB,),
        in_specs=[
            pl.BlockSpec((2 * NC, RB, HQ), lambda i: (0, i, 0)),
            pl.BlockSpec((RB, 1), lambda i: (i, 0)),
            pl.BlockSpec((1, HP), lambda i: (0, 0)),
            pl.BlockSpec((1, HP), lambda i: (0, 0)),
            pl.BlockSpec((1, HP), lambda i: (0, 0)),
        ],
        out_specs=[
            pl.BlockSpec((RB, HP), lambda i: (i, 0)),
            pl.BlockSpec((1, HP), lambda i: (0, 0)),
            pl.BlockSpec((1, HP), lambda i: (0, 0)),
        ],
        out_shape=[
            jax.ShapeDtypeStruct((NP, HP), F32),
            jax.ShapeDtypeStruct((1, HP), F32),
            jax.ShapeDtypeStruct((1, HP), F32),
        ],
        scratch_shapes=[pltpu.VMEM((1, HP), F32), pltpu.VMEM((1, HP), F32)],
    )(P, dinv, b, g, be)


def _tc_jk(us, sts, Wjk, bjk):
    def body(u0, u1, u2, u3, s0, t0, s1, t1, s2, t2, s3, t3,
             wjk_ref, bjk_ref, o_ref):
        acc = jnp.broadcast_to(bjk_ref[...], (RB, HP))
        for l, (u_ref, s_ref, t_ref) in enumerate(
                ((u0, s0, t0), (u1, s1, t1), (u2, s2, t2), (u3, s3, t3))):
            a = jnp.maximum(u_ref[...] * s_ref[...] + t_ref[...], 0.0)
            acc = acc + jax.lax.dot_general(
                a, wjk_ref[l], (((1,), (0,)), ((), ())),
                preferred_element_type=F32)
        o_ref[...] = acc

    in_specs = [pl.BlockSpec((RB, HP), lambda i: (i, 0))] * 4
    in_specs += [pl.BlockSpec((1, HP), lambda i: (0, 0))] * 8
    in_specs += [pl.BlockSpec((4, HP, HP), lambda i: (0, 0, 0)),
                 pl.BlockSpec((1, HP), lambda i: (0, 0))]
    args = list(us)
    for (s, t) in sts:
        args += [s, t]
    args += [Wjk, bjk]
    return pl.pallas_call(
        body,
        grid=(NRB,),
        in_specs=in_specs,
        out_specs=pl.BlockSpec((RB, HP), lambda i: (i, 0)),
        out_shape=jax.ShapeDtypeStruct((NP, HP), F32),
    )(*args)


def _tc_bounds(batch2):
    NB, BL = batch2.shape

    def body(b_ref, o_ref):
        idx = lax.broadcasted_iota(I32, (NW * LAN, 1), 0)
        th = (idx // LAN) * 8 + idx % LAN
        acc = jnp.zeros((NW * LAN, 1), I32)
        for k in range(NB):
            b = b_ref[pl.ds(k, 1), :]
            acc = acc + jnp.sum(jnp.where(b < th, 1, 0).astype(I32),
                                axis=1, keepdims=True)
        o_ref[...] = acc.reshape(NW, LAN)

    return pl.pallas_call(
        body,
        out_shape=jax.ShapeDtypeStruct((NW, LAN), I32),
    )(batch2)


def _tc_head(pooled, Wlin, blin):
    def body(p_ref, w_ref, b_ref, o_ref):
        o_ref[...] = jax.lax.dot_general(
            p_ref[...], w_ref[...], (((1,), (0,)), ((), ())),
            preferred_element_type=F32) + b_ref[...]

    return pl.pallas_call(
        body,
        out_shape=jax.ShapeDtypeStruct((G, 1), F32),
    )(pooled, Wlin, blin)


def _pad_rows(a, rows):
    return jnp.pad(a, ((0, rows - a.shape[0]), (0, 0)))


def _pad_feat(v):
    return jnp.pad(v.reshape(1, -1), ((0, 0), (0, HP - v.shape[0])))


def kernel(x, edge_index, batch, W0, b0, g0, be0, W1, b1, g1, be1,
           W2, b2, g2, be2, W3, b3, g3, be3, Wjk, bjk, Wlin, blin):
    src = edge_index[0].astype(I32)
    dst = edge_index[1].astype(I32)
    srcp = jnp.concatenate([src, jnp.zeros((EP - E,), I32)])
    dstp = jnp.concatenate([dst, jnp.full((EP - E,), N, I32)])
    srcp3 = srcp.reshape(NS, CPS, CH)
    dstp3 = dstp.reshape(NS, CPS, CH)

    xp = _pad_rows(x, NP)
    W0p = jnp.pad(W0, ((0, 0), (0, HP - H)))
    W1p = jnp.pad(W1, ((0, HP - H), (0, HP - H)))
    W2p = jnp.pad(W2, ((0, HP - H), (0, HP - H)))
    W3p = jnp.pad(W3, ((0, HP - H), (0, HP - H)))
    bs = [_pad_feat(b) for b in (b0, b1, b2, b3)]
    gs = [_pad_feat(g) for g in (g0, g1, g2, g3)]
    bes = [_pad_feat(b) for b in (be0, be1, be2, be3)]
    Wjkp = jnp.pad(Wjk.reshape(4, H, H), ((0, 0), (0, HP - H), (0, HP - H)))
    bjkp = _pad_feat(bjk)
    Wlinp = jnp.pad(Wlin, ((0, HP - H), (0, 0)))
    blinp = blin.reshape(1, 1)

    zeros_rows = jnp.zeros((CH, LAN), F32)
    ones_rows = jnp.ones((CH, LAN), F32)

    degp = _sc_degree(dstp, zeros_rows, ones_rows)
    m0q = _tc_matmul(xp, W0p)
    dinv, *hsq0 = _tc_dinv(degp, m0q)

    us, sts = [], []
    h_in, W_l, aff = None, None, None
    for l in range(4):
        if l == 0:
            hsq = hsq0
        else:
            hsq = _tc_matmul(h_in, W_l, dinv, affine=aff)
        P = _sc_aggregate(hsq, srcp3, dstp3)
        u, s_l, t_l = _tc_reduce(P, dinv, bs[l], gs[l], bes[l])
        us.append(u)
        sts.append((s_l, t_l))
        if l < 3:
            h_in, W_l, aff = u, (W1p, W2p, W3p)[l], sts[-1]

    hjk = _tc_jk(us, sts, Wjkp, bjkp)
    starts = _tc_bounds(batch.astype(I32).reshape(10, N // 10))
    pooled = _sc_segmax(hjk, starts)
    return _tc_head(pooled, Wlinp, blinp)
